# baseline (speedup 1.0000x reference)
import jax
import jax.numpy as jnp
from jax.experimental import pallas as pl
from jax.experimental.pallas import tpu as pltpu
from jax.experimental.pallas import tpu_sc as plsc

_ROWS = 8


def _sc_gather(t_hbm, sac_hbm, somac_hbm, c1_hbm, c2_hbm,
               t_s, sac_s, somac_s, c1_s, c2_s, sem):
    idx = jax.lax.axis_index("core")

    @pl.when(idx == 0)
    def _():
        pltpu.async_copy(t_hbm, t_s, sem).wait()
        pltpu.async_copy(sac_hbm, sac_s, sem).wait()
        pltpu.async_copy(somac_hbm, somac_s, sem).wait()

        @pl.loop(0, t_s.shape[0])
        def _(i):
            ti = t_s[i]
            c1_s[i] = sac_s[ti]
            c2_s[i] = somac_s[ti]

        pltpu.async_copy(c1_s, c1_hbm, sem).wait()
        pltpu.async_copy(c2_s, c2_hbm, sem).wait()


def _gather_coeffs(t, sac, somac):
    B = t.shape[0]
    S = sac.shape[0]
    mesh = plsc.ScalarSubcoreMesh(axis_name="core", num_cores=2)
    f = pl.kernel(
        _sc_gather,
        out_type=(
            jax.ShapeDtypeStruct((B,), jnp.float32),
            jax.ShapeDtypeStruct((B,), jnp.float32),
        ),
        mesh=mesh,
        scratch_types=[
            pltpu.SMEM((B,), jnp.int32),
            pltpu.SMEM((S,), jnp.float32),
            pltpu.SMEM((S,), jnp.float32),
            pltpu.SMEM((B,), jnp.float32),
            pltpu.SMEM((B,), jnp.float32),
            pltpu.SemaphoreType.DMA,
        ],
    )
    return f(t, sac, somac)


def _body(c1_ref, c2_ref, x_ref, n_ref, o_ref):
    i = pl.program_id(0)
    for r in range(_ROWS):
        c1 = c1_ref[i * _ROWS + r]
        c2 = c2_ref[i * _ROWS + r]
        o_ref[r, :, :] = c1 * x_ref[r, :, :] + c2 * n_ref[r, :, :]


def kernel(data_start, t, noise, sqrt_alphas_cumprod, sqrt_one_minus_alphas_cumprod):
    B, N, D = data_start.shape
    xt = jnp.transpose(data_start, (0, 2, 1))
    nt = jnp.transpose(noise, (0, 2, 1))

    c1, c2 = _gather_coeffs(t, sqrt_alphas_cumprod, sqrt_one_minus_alphas_cumprod)

    grid_spec = pltpu.PrefetchScalarGridSpec(
        num_scalar_prefetch=2,
        grid=(B // _ROWS,),
        in_specs=[
            pl.BlockSpec((_ROWS, D, N), lambda i, *_: (i, 0, 0)),
            pl.BlockSpec((_ROWS, D, N), lambda i, *_: (i, 0, 0)),
        ],
        out_specs=pl.BlockSpec((_ROWS, D, N), lambda i, *_: (i, 0, 0)),
    )

    out_t = pl.pallas_call(
        _body,
        grid_spec=grid_spec,
        out_shape=jax.ShapeDtypeStruct((B, D, N), jnp.float32),
    )(c1, c2, xt, nt)
    return jnp.transpose(out_t, (0, 2, 1))


# speedup vs baseline: 1.6330x; 1.6330x over previous
import jax
import jax.numpy as jnp
from jax.experimental import pallas as pl
from jax.experimental.pallas import tpu as pltpu

_CH = 4
_NBUF = 4


def _body(t_ref, sac_ref, somac_ref, x_hbm, n_hbm, o_hbm,
          xb, nb, ob, xsem, nsem, osem):
    B = x_hbm.shape[0]
    nchunks = B // _CH

    def in_copies(k):
        slot = k % _NBUF
        sl = pl.ds(k * _CH, _CH)
        return (
            pltpu.make_async_copy(x_hbm.at[sl], xb.at[slot], xsem.at[slot]),
            pltpu.make_async_copy(n_hbm.at[sl], nb.at[slot], nsem.at[slot]),
        )

    def out_copy(k):
        slot = k % _NBUF
        sl = pl.ds(k * _CH, _CH)
        return pltpu.make_async_copy(ob.at[slot], o_hbm.at[sl], osem.at[slot])

    for k in range(_NBUF):
        cx, cn = in_copies(k)
        cx.start()
        cn.start()

    for k in range(nchunks):
        slot = k % _NBUF
        cx, cn = in_copies(k)
        cx.wait()
        cn.wait()
        if k >= _NBUF:
            out_copy(k - _NBUF).wait()
        for r in range(_CH):
            tb = t_ref[k * _CH + r]
            c1 = sac_ref[tb]
            c2 = somac_ref[tb]
            ob[slot, r, :, :] = c1 * xb[slot, r, :, :] + c2 * nb[slot, r, :, :]
        out_copy(k).start()
        if k + _NBUF < nchunks:
            cx, cn = in_copies(k + _NBUF)
            cx.start()
            cn.start()

    for k in range(nchunks - _NBUF, nchunks):
        out_copy(k).wait()


def kernel(data_start, t, noise, sqrt_alphas_cumprod, sqrt_one_minus_alphas_cumprod):
    B, N, D = data_start.shape
    xt = jnp.transpose(data_start, (0, 2, 1))
    nt = jnp.transpose(noise, (0, 2, 1))

    grid_spec = pltpu.PrefetchScalarGridSpec(
        num_scalar_prefetch=3,
        grid=(1,),
        in_specs=[
            pl.BlockSpec(memory_space=pl.ANY),
            pl.BlockSpec(memory_space=pl.ANY),
        ],
        out_specs=pl.BlockSpec(memory_space=pl.ANY),
        scratch_shapes=[
            pltpu.VMEM((_NBUF, _CH, D, N), jnp.float32),
            pltpu.VMEM((_NBUF, _CH, D, N), jnp.float32),
            pltpu.VMEM((_NBUF, _CH, D, N), jnp.float32),
            pltpu.SemaphoreType.DMA((_NBUF,)),
            pltpu.SemaphoreType.DMA((_NBUF,)),
            pltpu.SemaphoreType.DMA((_NBUF,)),
        ],
    )

    out_t = pl.pallas_call(
        _body,
        grid_spec=grid_spec,
        out_shape=jax.ShapeDtypeStruct((B, D, N), jnp.float32),
    )(t, sqrt_alphas_cumprod, sqrt_one_minus_alphas_cumprod, xt, nt)
    return jnp.transpose(out_t, (0, 2, 1))


# speedup vs baseline: 1.6401x; 1.0044x over previous
import jax
import jax.numpy as jnp
from jax.experimental import pallas as pl
from jax.experimental.pallas import tpu as pltpu

_CH = 2
_NBUF = 8


def _body(t_ref, sac_ref, somac_ref, x_hbm, n_hbm, o_hbm,
          xb, nb, ob, xsem, nsem, osem):
    B = x_hbm.shape[0]
    nchunks = B // _CH

    def in_copies(k):
        slot = k % _NBUF
        sl = pl.ds(k * _CH, _CH)
        return (
            pltpu.make_async_copy(x_hbm.at[sl], xb.at[slot], xsem.at[slot]),
            pltpu.make_async_copy(n_hbm.at[sl], nb.at[slot], nsem.at[slot]),
        )

    def out_copy(k):
        slot = k % _NBUF
        sl = pl.ds(k * _CH, _CH)
        return pltpu.make_async_copy(ob.at[slot], o_hbm.at[sl], osem.at[slot])

    for k in range(_NBUF):
        cx, cn = in_copies(k)
        cx.start()
        cn.start()

    for k in range(nchunks):
        slot = k % _NBUF
        cx, cn = in_copies(k)
        cx.wait()
        cn.wait()
        if k >= _NBUF:
            out_copy(k - _NBUF).wait()
        for r in range(_CH):
            tb = t_ref[k * _CH + r]
            c1 = sac_ref[tb]
            c2 = somac_ref[tb]
            ob[slot, r, :, :] = c1 * xb[slot, r, :, :] + c2 * nb[slot, r, :, :]
        out_copy(k).start()
        if k + _NBUF < nchunks:
            cx, cn = in_copies(k + _NBUF)
            cx.start()
            cn.start()

    for k in range(nchunks - _NBUF, nchunks):
        out_copy(k).wait()


def kernel(data_start, t, noise, sqrt_alphas_cumprod, sqrt_one_minus_alphas_cumprod):
    B, N, D = data_start.shape
    xt = jnp.transpose(data_start, (0, 2, 1))
    nt = jnp.transpose(noise, (0, 2, 1))

    grid_spec = pltpu.PrefetchScalarGridSpec(
        num_scalar_prefetch=3,
        grid=(1,),
        in_specs=[
            pl.BlockSpec(memory_space=pl.ANY),
            pl.BlockSpec(memory_space=pl.ANY),
        ],
        out_specs=pl.BlockSpec(memory_space=pl.ANY),
        scratch_shapes=[
            pltpu.VMEM((_NBUF, _CH, D, N), jnp.float32),
            pltpu.VMEM((_NBUF, _CH, D, N), jnp.float32),
            pltpu.VMEM((_NBUF, _CH, D, N), jnp.float32),
            pltpu.SemaphoreType.DMA((_NBUF,)),
            pltpu.SemaphoreType.DMA((_NBUF,)),
            pltpu.SemaphoreType.DMA((_NBUF,)),
        ],
    )

    out_t = pl.pallas_call(
        _body,
        grid_spec=grid_spec,
        out_shape=jax.ShapeDtypeStruct((B, D, N), jnp.float32),
    )(t, sqrt_alphas_cumprod, sqrt_one_minus_alphas_cumprod, xt, nt)
    return jnp.transpose(out_t, (0, 2, 1))
